# baseline (device time: 121620 ns/iter reference)
import jax
import jax.numpy as jnp
from jax import lax
from jax.experimental import pallas as pl
from jax.experimental.pallas import tpu as pltpu

N_DEV = 16
N_TOK = 2048
D_MODEL = 512
N_EXP = 128
D_FF = 1024
E_LOC = N_EXP // N_DEV
CHUNK = N_TOK // N_DEV
R_STEPS = 8
L_STEPS = 7
N_SEMS = 2 * (R_STEPS + L_STEPS)


def kernel(x, router_W, route_idx, expert_W, shared_W):
    def body(x_ref, rw_ref, idx_ref, ew_ref, sw_ref, out_ref,
             acc_ref, rs_ref, p_ref, send_sems, recv_sems):
        my = lax.axis_index("i")
        left = jnp.mod(my - 1, N_DEV)
        right = jnp.mod(my + 1, N_DEV)

        barrier_sem = pltpu.get_barrier_semaphore()
        pl.semaphore_signal(barrier_sem, inc=1, device_id=(left,),
                            device_id_type=pl.DeviceIdType.MESH)
        pl.semaphore_signal(barrier_sem, inc=1, device_id=(right,),
                            device_id_type=pl.DeviceIdType.MESH)
        pl.semaphore_wait(barrier_sem, 2)

        pending = []

        def copy(src, dst, sem_idx, dev):
            rdma = pltpu.make_async_remote_copy(
                src_ref=src, dst_ref=dst,
                send_sem=send_sems.at[sem_idx],
                recv_sem=recv_sems.at[sem_idx],
                device_id=(dev,), device_id_type=pl.DeviceIdType.MESH,
            )
            rdma.start()
            pending.append(rdma)
            return rdma

        x32 = x_ref[:, :]
        scores = jnp.dot(x32, rw_ref[:, :], preferred_element_type=jnp.float32)
        s_max = jnp.max(scores, axis=-1, keepdims=True)
        e = jnp.exp(scores - s_max)
        probs = e / jnp.sum(e, axis=-1, keepdims=True)
        idx = idx_ref[:, :]
        e_iota = lax.broadcasted_iota(jnp.int32, (N_TOK, N_EXP), 1)
        p_tok = jnp.sum(jnp.where(e_iota == idx, probs, 0.0),
                        axis=1, keepdims=True)
        p_ref[:, :] = p_tok.astype(jnp.bfloat16)

        def compute_chunk(c):
            rows = pl.ds(c * CHUNK, CHUNK)
            xc = x_ref[rows, :].astype(jnp.bfloat16)
            ic = idx_ref[rows, :]
            pc = p_ref[rows, :]
            accc = jnp.zeros((CHUNK, D_FF), jnp.float32)
            for j in range(E_LOC):
                e_g = my * E_LOC + j
                wj = jnp.where(ic == e_g, pc, jnp.bfloat16(0.0))
                accc = accc + jnp.dot(xc * wj, ew_ref[j].astype(jnp.bfloat16),
                                      preferred_element_type=jnp.float32)
            acc_ref[c] = accc.astype(jnp.bfloat16)

        compute_chunk(jnp.mod(my + 8, N_DEV))
        compute_chunk(jnp.mod(my - 7, N_DEV))

        for h in range(1, R_STEPS + 1):
            r_rd = copy(acc_ref.at[jnp.mod(my + 9 - h, N_DEV)],
                        rs_ref.at[h - 1], h - 1, right)
            l_rd = None
            if h <= L_STEPS:
                l_rd = copy(acc_ref.at[jnp.mod(my - 8 + h, N_DEV)],
                            rs_ref.at[R_STEPS + h - 1], R_STEPS + h - 1, left)
            if h <= L_STEPS:
                compute_chunk(jnp.mod(my + 8 - h, N_DEV))
                compute_chunk(jnp.mod(my - 7 + h, N_DEV))
            else:
                pass
            r_rd.wait_recv()
            rr = jnp.mod(my + 8 - h, N_DEV)
            acc_ref[rr] = acc_ref[rr] + rs_ref[h - 1]
            if l_rd is not None:
                l_rd.wait_recv()
                rl = jnp.mod(my - 7 + h, N_DEV)
                acc_ref[rl] = acc_ref[rl] + rs_ref[R_STEPS + h - 1]

        my_rows = pl.ds(my * CHUNK, CHUNK)
        xo = x_ref[my_rows, :].astype(jnp.bfloat16)
        shared_o = jnp.dot(xo, sw_ref[:, :].astype(jnp.bfloat16),
                           preferred_element_type=jnp.float32)
        out_ref[my_rows, :] = (acc_ref[my].astype(jnp.float32)
                               + shared_o).astype(jnp.bfloat16)

        ag_base = R_STEPS + L_STEPS
        for h in range(1, R_STEPS + 1):
            sr = jnp.mod(my - h + 1, N_DEV)
            r_rd = copy(out_ref.at[pl.ds(sr * CHUNK, CHUNK), :],
                        out_ref.at[pl.ds(sr * CHUNK, CHUNK), :],
                        ag_base + h - 1, right)
            l_rd = None
            if h <= L_STEPS:
                sl = jnp.mod(my + h - 1, N_DEV)
                l_rd = copy(out_ref.at[pl.ds(sl * CHUNK, CHUNK), :],
                            out_ref.at[pl.ds(sl * CHUNK, CHUNK), :],
                            ag_base + R_STEPS + h - 1, left)
            r_rd.wait_recv()
            if l_rd is not None:
                l_rd.wait_recv()

        for rdma in pending:
            rdma.wait_send()

    return pl.pallas_call(
        body,
        out_shape=jax.ShapeDtypeStruct((N_TOK, D_FF), jnp.bfloat16),
        in_specs=[pl.BlockSpec(memory_space=pltpu.VMEM)] * 5,
        out_specs=pl.BlockSpec(memory_space=pltpu.VMEM),
        scratch_shapes=[
            pltpu.VMEM((N_DEV, CHUNK, D_FF), jnp.bfloat16),
            pltpu.VMEM((R_STEPS + L_STEPS, CHUNK, D_FF), jnp.bfloat16),
            pltpu.VMEM((N_TOK, 1), jnp.bfloat16),
            pltpu.SemaphoreType.DMA((N_SEMS,)),
            pltpu.SemaphoreType.DMA((N_SEMS,)),
        ],
        compiler_params=pltpu.CompilerParams(
            collective_id=0,
            vmem_limit_bytes=100 * 1024 * 1024,
        ),
    )(x, router_W, route_idx, expert_W, shared_W)


# device time: 103721 ns/iter; 1.1726x vs baseline; 1.1726x over previous
import jax
import jax.numpy as jnp
from jax import lax
from jax.experimental import pallas as pl
from jax.experimental.pallas import tpu as pltpu

N_DEV = 16
N_TOK = 2048
D_MODEL = 512
N_EXP = 128
D_FF = 1024
E_LOC = N_EXP // N_DEV
CHUNK = N_TOK // N_DEV
R_STEPS = 8
L_STEPS = 7
N_SEMS = 2 * (R_STEPS + L_STEPS)

RING = (0, 4, 8, 12, 13, 9, 5, 1, 2, 6, 10, 14, 15, 11, 7, 3)


def kernel(x, router_W, route_idx, expert_W, shared_W):
    ring = jnp.array(RING, dtype=jnp.int32)
    my_l = lax.axis_index("i")
    rpos = jnp.argmax((ring == my_l).astype(jnp.int32)).astype(jnp.int32)
    right_l = ring[jnp.mod(rpos + 1, N_DEV)]
    left_l = ring[jnp.mod(rpos - 1, N_DEV)]
    scalars = jnp.stack([rpos, left_l, right_l]).astype(jnp.int32)

    def body(scal_ref, x_ref, rw_ref, idx_ref, ew_ref, sw_ref, out_ref,
             acc_ref, rs_ref, p_ref, send_sems, recv_sems):
        my = scal_ref[0]
        left = scal_ref[1]
        right = scal_ref[2]
        my_log = lax.axis_index("i")

        barrier_sem = pltpu.get_barrier_semaphore()
        pl.semaphore_signal(barrier_sem, inc=1, device_id=(left,),
                            device_id_type=pl.DeviceIdType.MESH)
        pl.semaphore_signal(barrier_sem, inc=1, device_id=(right,),
                            device_id_type=pl.DeviceIdType.MESH)
        pl.semaphore_wait(barrier_sem, 2)

        pending = []

        def copy(src, dst, sem_idx, dev):
            rdma = pltpu.make_async_remote_copy(
                src_ref=src, dst_ref=dst,
                send_sem=send_sems.at[sem_idx],
                recv_sem=recv_sems.at[sem_idx],
                device_id=(dev,), device_id_type=pl.DeviceIdType.MESH,
            )
            rdma.start()
            pending.append(rdma)
            return rdma

        x32 = x_ref[:, :]
        scores = jnp.dot(x32, rw_ref[:, :], preferred_element_type=jnp.float32)
        s_max = jnp.max(scores, axis=-1, keepdims=True)
        e = jnp.exp(scores - s_max)
        probs = e / jnp.sum(e, axis=-1, keepdims=True)
        idx = idx_ref[:, :]
        e_iota = lax.broadcasted_iota(jnp.int32, (N_TOK, N_EXP), 1)
        p_tok = jnp.sum(jnp.where(e_iota == idx, probs, 0.0),
                        axis=1, keepdims=True)
        p_ref[:, :] = p_tok.astype(jnp.bfloat16)

        def compute_chunk(c):
            rows = pl.ds(c * CHUNK, CHUNK)
            xc = x_ref[rows, :].astype(jnp.bfloat16)
            ic = idx_ref[rows, :]
            pc = p_ref[rows, :]
            accc = jnp.zeros((CHUNK, D_FF), jnp.float32)
            for j in range(E_LOC):
                e_g = my_log * E_LOC + j
                wj = jnp.where(ic == e_g, pc, jnp.bfloat16(0.0))
                accc = accc + jnp.dot(xc * wj, ew_ref[j].astype(jnp.bfloat16),
                                      preferred_element_type=jnp.float32)
            acc_ref[c] = accc.astype(jnp.bfloat16)

        compute_chunk(jnp.mod(my + 8, N_DEV))
        compute_chunk(jnp.mod(my - 7, N_DEV))

        for h in range(1, R_STEPS + 1):
            r_rd = copy(acc_ref.at[jnp.mod(my + 9 - h, N_DEV)],
                        rs_ref.at[h - 1], h - 1, right)
            l_rd = None
            if h <= L_STEPS:
                l_rd = copy(acc_ref.at[jnp.mod(my - 8 + h, N_DEV)],
                            rs_ref.at[R_STEPS + h - 1], R_STEPS + h - 1, left)
            if h <= L_STEPS:
                compute_chunk(jnp.mod(my + 8 - h, N_DEV))
                compute_chunk(jnp.mod(my - 7 + h, N_DEV))
            else:
                pass
            r_rd.wait_recv()
            rr = jnp.mod(my + 8 - h, N_DEV)
            acc_ref[rr] = acc_ref[rr] + rs_ref[h - 1]
            if l_rd is not None:
                l_rd.wait_recv()
                rl = jnp.mod(my - 7 + h, N_DEV)
                acc_ref[rl] = acc_ref[rl] + rs_ref[R_STEPS + h - 1]

        my_rows = pl.ds(my * CHUNK, CHUNK)
        xo = x_ref[my_rows, :].astype(jnp.bfloat16)
        shared_o = jnp.dot(xo, sw_ref[:, :].astype(jnp.bfloat16),
                           preferred_element_type=jnp.float32)
        out_ref[my_rows, :] = (acc_ref[my].astype(jnp.float32)
                               + shared_o).astype(jnp.bfloat16)

        ag_base = R_STEPS + L_STEPS
        for h in range(1, R_STEPS + 1):
            sr = jnp.mod(my - h + 1, N_DEV)
            r_rd = copy(out_ref.at[pl.ds(sr * CHUNK, CHUNK), :],
                        out_ref.at[pl.ds(sr * CHUNK, CHUNK), :],
                        ag_base + h - 1, right)
            l_rd = None
            if h <= L_STEPS:
                sl = jnp.mod(my + h - 1, N_DEV)
                l_rd = copy(out_ref.at[pl.ds(sl * CHUNK, CHUNK), :],
                            out_ref.at[pl.ds(sl * CHUNK, CHUNK), :],
                            ag_base + R_STEPS + h - 1, left)
            r_rd.wait_recv()
            if l_rd is not None:
                l_rd.wait_recv()

        for rdma in pending:
            rdma.wait_send()

    return pl.pallas_call(
        body,
        out_shape=jax.ShapeDtypeStruct((N_TOK, D_FF), jnp.bfloat16),
        in_specs=[pl.BlockSpec(memory_space=pltpu.SMEM)]
        + [pl.BlockSpec(memory_space=pltpu.VMEM)] * 5,
        out_specs=pl.BlockSpec(memory_space=pltpu.VMEM),
        scratch_shapes=[
            pltpu.VMEM((N_DEV, CHUNK, D_FF), jnp.bfloat16),
            pltpu.VMEM((R_STEPS + L_STEPS, CHUNK, D_FF), jnp.bfloat16),
            pltpu.VMEM((N_TOK, 1), jnp.bfloat16),
            pltpu.SemaphoreType.DMA((N_SEMS,)),
            pltpu.SemaphoreType.DMA((N_SEMS,)),
        ],
        compiler_params=pltpu.CompilerParams(
            collective_id=0,
            vmem_limit_bytes=100 * 1024 * 1024,
        ),
    )(scalars, x, router_W, route_idx, expert_W, shared_W)


# device time: 96327 ns/iter; 1.2626x vs baseline; 1.0768x over previous
import jax
import jax.numpy as jnp
from jax import lax
from jax.experimental import pallas as pl
from jax.experimental.pallas import tpu as pltpu

N_DEV = 16
N_TOK = 2048
D_MODEL = 512
N_EXP = 128
D_FF = 1024
E_LOC = N_EXP // N_DEV
CHUNK = N_TOK // N_DEV
HALF = CHUNK // 2
R_STEPS = 8
L_STEPS = 7
AGR_BASE = 15
AGL_BASE = 31
N_SEMS = 45

RING = (0, 4, 8, 12, 13, 9, 5, 1, 2, 6, 10, 14, 15, 11, 7, 3)


def kernel(x, router_W, route_idx, expert_W, shared_W):
    ring = jnp.array(RING, dtype=jnp.int32)
    my_l = lax.axis_index("i")
    rpos = jnp.argmax((ring == my_l).astype(jnp.int32)).astype(jnp.int32)
    right_l = ring[jnp.mod(rpos + 1, N_DEV)]
    left_l = ring[jnp.mod(rpos - 1, N_DEV)]
    scalars = jnp.stack([rpos, left_l, right_l]).astype(jnp.int32)

    def body(scal_ref, x_ref, rw_ref, idx_ref, ew_ref, sw_ref, out_ref,
             acc_ref, rs_ref, p_ref, xb_ref, ewb_ref, send_sems, recv_sems):
        my = scal_ref[0]
        left = scal_ref[1]
        right = scal_ref[2]
        my_log = lax.axis_index("i")

        barrier_sem = pltpu.get_barrier_semaphore()
        pl.semaphore_signal(barrier_sem, inc=1, device_id=(left,),
                            device_id_type=pl.DeviceIdType.MESH)
        pl.semaphore_signal(barrier_sem, inc=1, device_id=(right,),
                            device_id_type=pl.DeviceIdType.MESH)
        pl.semaphore_wait(barrier_sem, 2)

        pending = []

        def copy(src, dst, sem_idx, dev):
            rdma = pltpu.make_async_remote_copy(
                src_ref=src, dst_ref=dst,
                send_sem=send_sems.at[sem_idx],
                recv_sem=recv_sems.at[sem_idx],
                device_id=(dev,), device_id_type=pl.DeviceIdType.MESH,
            )
            rdma.start()
            pending.append(rdma)
            return rdma

        xb_ref[:, :] = x_ref[:, :].astype(jnp.bfloat16)
        for j in range(E_LOC):
            ewb_ref[j] = ew_ref[j].astype(jnp.bfloat16)

        scores = jnp.dot(xb_ref[:, :], rw_ref[:, :].astype(jnp.bfloat16),
                         preferred_element_type=jnp.float32)
        s_max = jnp.max(scores, axis=-1, keepdims=True)
        e = jnp.exp(scores - s_max)
        probs = e / jnp.sum(e, axis=-1, keepdims=True)
        idx = idx_ref[:, :]
        e_iota = lax.broadcasted_iota(jnp.int32, (N_TOK, N_EXP), 1)
        p_tok = jnp.sum(jnp.where(e_iota == idx, probs, 0.0),
                        axis=1, keepdims=True)
        p_ref[:, :] = p_tok.astype(jnp.bfloat16)

        def compute_pair(a, b):
            ra = pl.ds(a * CHUNK, CHUNK)
            rb = pl.ds(b * CHUNK, CHUNK)
            xc = jnp.concatenate([xb_ref[ra, :], xb_ref[rb, :]], axis=0)
            ic = jnp.concatenate([idx_ref[ra, :], idx_ref[rb, :]], axis=0)
            pc = jnp.concatenate([p_ref[ra, :], p_ref[rb, :]], axis=0)
            accc = jnp.zeros((2 * CHUNK, D_FF), jnp.float32)
            for j in range(E_LOC):
                e_g = my_log * E_LOC + j
                wj = jnp.where(ic == e_g, pc, jnp.bfloat16(0.0))
                accc = accc + jnp.dot(xc * wj, ewb_ref[j],
                                      preferred_element_type=jnp.float32)
            accb = accc.astype(jnp.bfloat16)
            acc_ref[a] = accb[:CHUNK]
            acc_ref[b] = accb[CHUNK:]

        compute_pair(jnp.mod(my + 8, N_DEV), jnp.mod(my - 7, N_DEV))

        for h in range(1, R_STEPS + 1):
            r_rd = copy(acc_ref.at[jnp.mod(my + 9 - h, N_DEV)],
                        rs_ref.at[h - 1], h - 1, right)
            l_rd = None
            if h <= L_STEPS:
                l_rd = copy(acc_ref.at[jnp.mod(my - 8 + h, N_DEV)],
                            rs_ref.at[R_STEPS + h - 1], R_STEPS + h - 1, left)
            if h <= L_STEPS:
                compute_pair(jnp.mod(my + 8 - h, N_DEV),
                             jnp.mod(my - 7 + h, N_DEV))
            r_rd.wait_recv()
            rr = jnp.mod(my + 8 - h, N_DEV)
            acc_ref[rr] = acc_ref[rr] + rs_ref[h - 1]
            if l_rd is not None:
                l_rd.wait_recv()
                rl = jnp.mod(my - 7 + h, N_DEV)
                acc_ref[rl] = acc_ref[rl] + rs_ref[R_STEPS + h - 1]

        my_rows = pl.ds(my * CHUNK, CHUNK)
        shared_o = jnp.dot(xb_ref[my_rows, :],
                           sw_ref[:, :].astype(jnp.bfloat16),
                           preferred_element_type=jnp.float32)
        out_ref[my_rows, :] = (acc_ref[my].astype(jnp.float32)
                               + shared_o).astype(jnp.bfloat16)

        def ag_copy(chunk_idx, half, sem_idx, dev):
            rows = pl.ds(chunk_idx * CHUNK + half * HALF, HALF)
            return copy(out_ref.at[rows, :], out_ref.at[rows, :],
                        sem_idx, dev)

        ag_r = {}
        ag_l = {}
        for k in range(2):
            ag_r[(1, k)] = ag_copy(my, k, AGR_BASE + k, right)
            ag_l[(1, k)] = ag_copy(my, k, AGL_BASE + k, left)
        for h in range(2, R_STEPS + 1):
            sr = jnp.mod(my - h + 1, N_DEV)
            for k in range(2):
                ag_r[(h - 1, k)].wait_recv()
                ag_r[(h, k)] = ag_copy(sr, k, AGR_BASE + 2 * (h - 1) + k,
                                       right)
            if h <= L_STEPS:
                sl = jnp.mod(my + h - 1, N_DEV)
                for k in range(2):
                    ag_l[(h - 1, k)].wait_recv()
                    ag_l[(h, k)] = ag_copy(sl, k, AGL_BASE + 2 * (h - 1) + k,
                                           left)
        for k in range(2):
            ag_r[(R_STEPS, k)].wait_recv()
            ag_l[(L_STEPS, k)].wait_recv()

        for rdma in pending:
            rdma.wait_send()

    return pl.pallas_call(
        body,
        out_shape=jax.ShapeDtypeStruct((N_TOK, D_FF), jnp.bfloat16),
        in_specs=[pl.BlockSpec(memory_space=pltpu.SMEM)]
        + [pl.BlockSpec(memory_space=pltpu.VMEM)] * 5,
        out_specs=pl.BlockSpec(memory_space=pltpu.VMEM),
        scratch_shapes=[
            pltpu.VMEM((N_DEV, CHUNK, D_FF), jnp.bfloat16),
            pltpu.VMEM((R_STEPS + L_STEPS, CHUNK, D_FF), jnp.bfloat16),
            pltpu.VMEM((N_TOK, 1), jnp.bfloat16),
            pltpu.VMEM((N_TOK, D_MODEL), jnp.bfloat16),
            pltpu.VMEM((E_LOC, D_MODEL, D_FF), jnp.bfloat16),
            pltpu.SemaphoreType.DMA((N_SEMS,)),
            pltpu.SemaphoreType.DMA((N_SEMS,)),
        ],
        compiler_params=pltpu.CompilerParams(
            collective_id=0,
            vmem_limit_bytes=128 * 1024 * 1024,
        ),
    )(scalars, x, router_W, route_idx, expert_W, shared_W)
